# baseline (speedup 1.0000x reference)
import functools
import math

import jax
import jax.numpy as jnp
from jax import lax
from jax.experimental import pallas as pl
from jax.experimental.pallas import tpu as pltpu
from jax.experimental.pallas import tpu_sc as plsc

NUM_CORES = 2
NUM_SUBCORES = 16
NW = NUM_CORES * NUM_SUBCORES
CHUNK = 128
LANES = 16


@functools.cache
def _build(B, D):
    b_per_w = B // NW
    n_chunks = b_per_w // CHUNK
    scale = math.sqrt(D)
    mesh = plsc.VectorSubcoreMesh(core_axis_name="c", subcore_axis_name="s")

    @functools.partial(
        pl.kernel,
        mesh=mesh,
        out_type=jax.ShapeDtypeStruct((NW, n_chunks, CHUNK, D), jnp.float32),
        scratch_types=[
            pltpu.VMEM((n_chunks, CHUNK), jnp.int32),
            pltpu.VMEM((CHUNK, D), jnp.float32),
            pltpu.SemaphoreType.DMA,
        ],
        compiler_params=pltpu.CompilerParams(use_tc_tiling_on_sc=False),
    )
    def k(x_hbm, lut_hbm, out_hbm, idx_v, rows_v, sem):
        wid = lax.axis_index("s") * NUM_CORES + lax.axis_index("c")
        pltpu.sync_copy(x_hbm.at[wid], idx_v)

        def chunk_body(ci, carry):
            pltpu.async_copy(lut_hbm.at[idx_v.at[ci]], rows_v, sem).wait()

            def row_body(ri, carry2):
                for j in range(D // LANES):
                    sl = (ri, pl.ds(j * LANES, LANES))
                    rows_v[sl] = rows_v[sl] * scale
                return carry2

            lax.fori_loop(0, CHUNK, row_body, 0)
            pltpu.sync_copy(rows_v, out_hbm.at[wid, ci])
            return carry

        lax.fori_loop(0, n_chunks, chunk_body, 0)

    return k


def kernel(x, lut):
    D = lut.shape[1]
    xf = x.reshape(-1).astype(jnp.int32)
    B = xf.shape[0]
    xf = xf.reshape(NW, B // NW // CHUNK, CHUNK)
    out = _build(B, D)(xf, lut)
    return out.reshape(x.shape + (D,))


# speedup vs baseline: 1.0821x; 1.0821x over previous
import functools
import math

import jax
import jax.numpy as jnp
from jax import lax
from jax.experimental import pallas as pl
from jax.experimental.pallas import tpu as pltpu
from jax.experimental.pallas import tpu_sc as plsc

NUM_CORES = 2
NUM_SUBCORES = 16
NW = NUM_CORES * NUM_SUBCORES
CHUNK = 128
LANES = 16
NBUF = 5


@functools.cache
def _build(B, D):
    b_per_w = B // NW
    n_chunks = b_per_w // CHUNK
    n_outer = n_chunks // NBUF
    scale = math.sqrt(D)
    mesh = plsc.VectorSubcoreMesh(core_axis_name="c", subcore_axis_name="s")

    @functools.partial(
        pl.kernel,
        mesh=mesh,
        out_type=jax.ShapeDtypeStruct((NW, n_chunks, CHUNK, D), jnp.float32),
        scratch_types=[
            pltpu.VMEM((n_chunks, CHUNK), jnp.int32),
            pltpu.VMEM((NBUF, CHUNK, D), jnp.float32),
            pltpu.SemaphoreType.DMA((NBUF,)),
            pltpu.SemaphoreType.DMA((NBUF,)),
        ],
        compiler_params=pltpu.CompilerParams(use_tc_tiling_on_sc=False),
    )
    def k(x_hbm, lut_hbm, out_hbm, idx_v, rows_v, gsem, ssem):
        wid = lax.axis_index("s") * NUM_CORES + lax.axis_index("c")
        pltpu.sync_copy(x_hbm.at[wid], idx_v)

        def start_gather(ci, b):
            pltpu.async_copy(lut_hbm.at[idx_v.at[ci]], rows_v.at[b], gsem.at[b])

        def wait_gather(ci, b):
            pltpu.make_async_copy(
                lut_hbm.at[idx_v.at[ci]], rows_v.at[b], gsem.at[b]
            ).wait()

        def start_scatter(ci, b):
            pltpu.async_copy(rows_v.at[b], out_hbm.at[wid, ci], ssem.at[b])

        def wait_scatter(ci, b):
            pltpu.make_async_copy(
                rows_v.at[b], out_hbm.at[wid, ci], ssem.at[b]
            ).wait()

        def scale_buf(b):
            @plsc.parallel_loop(0, CHUNK, unroll=8)
            def _(ri):
                for j in range(D // LANES):
                    sl = (b, ri, pl.ds(j * LANES, LANES))
                    rows_v[sl] = rows_v[sl] * scale

        for b in range(NBUF - 1):
            start_gather(b, b)

        @pl.loop(0, n_outer)
        def _(g):
            for b in range(NBUF):
                ci = g * NBUF + b
                bp = (b - 1) % NBUF
                cp = ci + NBUF - 1
                wait_gather(ci, b)
                scale_buf(b)
                if b == 0:
                    @pl.when(g > 0)
                    def _():
                        wait_scatter(ci - 1, bp)
                else:
                    wait_scatter(ci - 1, bp)

                @pl.when(cp < n_chunks)
                def _():
                    start_gather(cp, bp)

                start_scatter(ci, b)

        wait_scatter(n_chunks - 1, (n_chunks - 1) % NBUF)

    return k


def kernel(x, lut):
    D = lut.shape[1]
    xf = x.reshape(-1).astype(jnp.int32)
    B = xf.shape[0]
    xf = xf.reshape(NW, B // NW // CHUNK, CHUNK)
    out = _build(B, D)(xf, lut)
    return out.reshape(x.shape + (D,))
